# baseline (speedup 1.0000x reference)
import functools

import jax
import jax.numpy as jnp
from jax.experimental import pallas as pl
from jax.experimental.pallas import tpu as pltpu


def _fused_kernel(x_ref, w_ref, g_ref, o_ref, conv_sc, st_sc, *,
                  cin, cout, h, w, bi, bo, n_conv, inv_m, eps):
    t = pl.program_id(0)
    hw = h * w

    @pl.when(t == 0)
    def _():
        st_sc[...] = jnp.zeros_like(st_sc)

    @pl.when(t < n_conv)
    def _():
        wmod = jax.lax.broadcasted_iota(jnp.int32, (1, hw), 1) % w
        zcol = jnp.zeros((cin, 1), jnp.bfloat16)
        zrow = jnp.zeros((cout, w), jnp.float32)
        tsum = jnp.zeros((cout, 1), jnp.float32)
        tsq = jnp.zeros((cout, 1), jnp.float32)
        for i in range(bi):
            x = x_ref[i].astype(jnp.bfloat16)
            xr = jnp.where(wmod != 0,
                           jnp.concatenate([zcol, x[:, : hw - 1]], axis=1), 0)
            xl = jnp.where(wmod != w - 1,
                           jnp.concatenate([x[:, 1:], zcol], axis=1), 0)
            xw = jnp.concatenate([xr, x, xl], axis=0)

            p0 = jnp.dot(w_ref[0], xw, preferred_element_type=jnp.float32)
            p1 = jnp.dot(w_ref[1], xw, preferred_element_type=jnp.float32)
            p2 = jnp.dot(w_ref[2], xw, preferred_element_type=jnp.float32)

            acc = p1
            acc = acc + jnp.concatenate([zrow, p0[:, : hw - w]], axis=1)
            acc = acc + jnp.concatenate([p2[:, w:], zrow], axis=1)

            conv_sc[t * bi + i] = acc.astype(jnp.bfloat16)
            tsum = tsum + jnp.sum(acc, axis=1, keepdims=True)
            tsq = tsq + jnp.sum(acc * acc, axis=1, keepdims=True)
        st_sc[:, 0:1] = st_sc[:, 0:1] + tsum
        st_sc[:, 1:2] = st_sc[:, 1:2] + tsq

    @pl.when(t >= n_conv)
    def _():
        b = t - n_conv
        st = st_sc[...]
        mean = st[:, 0:1] * inv_m
        var = jnp.maximum(st[:, 1:2] * inv_m - mean * mean, 0.0)
        inv_std = jax.lax.rsqrt(var + eps)
        g = g_ref[...]
        scale = g[:, 0:1] * inv_std
        shift = g[:, 1:2] - mean * scale
        for i in range(bo):
            c = conv_sc[b * bo + i].astype(jnp.float32)
            o_ref[i] = jnp.maximum(c * scale + shift, 0.0)


@jax.jit
def kernel(x, conv_w, conv_b, gamma, beta, bn_w, bn_b, A_taskpair):
    del conv_b
    N, Cin, H, W = x.shape
    Cout = conv_w.shape[0]
    HW = H * W
    M = N * HW

    A = A_taskpair.astype(jnp.float32)
    gamma_eff = (A @ gamma.astype(jnp.float32).T).reshape(Cout)
    beta_eff = (A @ beta.astype(jnp.float32).T).reshape(Cout)
    g1 = bn_w.astype(jnp.float32) * gamma_eff
    g2 = bn_b.astype(jnp.float32) * gamma_eff + beta_eff
    g_pack = jnp.zeros((Cout, 8), jnp.float32)
    g_pack = g_pack.at[:, 0].set(g1).at[:, 1].set(g2)

    x_r = x.reshape(N, Cin, HW)
    w_cat = jnp.transpose(conv_w.astype(jnp.bfloat16), (2, 0, 3, 1)).reshape(
        3, Cout, 3 * Cin)

    BI = 4
    BO = 8
    n_conv = N // BI
    n_apply = N // BO
    kfn = functools.partial(
        _fused_kernel, cin=Cin, cout=Cout, h=H, w=W, bi=BI, bo=BO,
        n_conv=n_conv, inv_m=1.0 / float(M), eps=1e-5)
    out = pl.pallas_call(
        kfn,
        out_shape=jax.ShapeDtypeStruct((N, Cout, HW), jnp.float32),
        grid=(n_conv + n_apply,),
        in_specs=[
            pl.BlockSpec((BI, Cin, HW),
                         lambda t: (jnp.minimum(t, n_conv - 1), 0, 0)),
            pl.BlockSpec((3, Cout, 3 * Cin), lambda t: (0, 0, 0)),
            pl.BlockSpec((Cout, 8), lambda t: (0, 0)),
        ],
        out_specs=pl.BlockSpec(
            (BO, Cout, HW), lambda t: (jnp.maximum(t - n_conv, 0), 0, 0)),
        scratch_shapes=[
            pltpu.VMEM((N, Cout, HW), jnp.bfloat16),
            pltpu.VMEM((Cout, 8), jnp.float32),
        ],
        compiler_params=pltpu.CompilerParams(
            dimension_semantics=("arbitrary",)),
    )(x_r, w_cat, g_pack)

    return out.reshape(N, Cout, H, W)


# speedup vs baseline: 1.0095x; 1.0095x over previous
import functools

import jax
import jax.numpy as jnp
from jax.experimental import pallas as pl
from jax.experimental.pallas import tpu as pltpu


def _fused_kernel(x_ref, w_ref, g_ref, o_ref, conv_sc, st_sc, *,
                  cin, cout, h, w, bi, bo, n_conv, inv_m, eps):
    t = pl.program_id(0)
    hw = h * w

    @pl.when(t == 0)
    def _():
        st_sc[...] = jnp.zeros_like(st_sc)

    @pl.when(t < n_conv)
    def _():
        wmod = jax.lax.broadcasted_iota(jnp.int32, (1, hw), 1) % w
        zcol = jnp.zeros((cin, 1), jnp.bfloat16)
        zrow = jnp.zeros((cout, w), jnp.float32)
        tsum = jnp.zeros((cout, 1), jnp.float32)
        tsq = jnp.zeros((cout, 1), jnp.float32)
        for i in range(bi):
            x = x_ref[i].astype(jnp.bfloat16)
            xr = jnp.where(wmod != 0,
                           jnp.concatenate([zcol, x[:, : hw - 1]], axis=1), 0)
            xl = jnp.where(wmod != w - 1,
                           jnp.concatenate([x[:, 1:], zcol], axis=1), 0)
            xw = jnp.concatenate([xr, x, xl], axis=0)

            p0 = jnp.dot(w_ref[0], xw, preferred_element_type=jnp.float32)
            p1 = jnp.dot(w_ref[1], xw, preferred_element_type=jnp.float32)
            p2 = jnp.dot(w_ref[2], xw, preferred_element_type=jnp.float32)

            acc = p1
            acc = acc + jnp.concatenate([zrow, p0[:, : hw - w]], axis=1)
            acc = acc + jnp.concatenate([p2[:, w:], zrow], axis=1)

            conv_sc[t * bi + i] = acc.astype(jnp.bfloat16)
            tsum = tsum + jnp.sum(acc, axis=1, keepdims=True)
            tsq = tsq + jnp.sum(acc * acc, axis=1, keepdims=True)
        st_sc[:, 0:1] = st_sc[:, 0:1] + tsum
        st_sc[:, 1:2] = st_sc[:, 1:2] + tsq

    @pl.when(t >= n_conv)
    def _():
        b = t - n_conv
        st = st_sc[...]
        mean = st[:, 0:1] * inv_m
        var = jnp.maximum(st[:, 1:2] * inv_m - mean * mean, 0.0)
        inv_std = jax.lax.rsqrt(var + eps)
        g = g_ref[...]
        scale = g[:, 0:1] * inv_std
        shift = g[:, 1:2] - mean * scale
        for i in range(bo):
            c = conv_sc[b * bo + i].astype(jnp.float32)
            o_ref[i] = jnp.maximum(c * scale + shift, 0.0)


@jax.jit
def kernel(x, conv_w, conv_b, gamma, beta, bn_w, bn_b, A_taskpair):
    del conv_b
    N, Cin, H, W = x.shape
    Cout = conv_w.shape[0]
    HW = H * W
    M = N * HW

    A = A_taskpair.astype(jnp.float32)
    gamma_eff = (A @ gamma.astype(jnp.float32).T).reshape(Cout)
    beta_eff = (A @ beta.astype(jnp.float32).T).reshape(Cout)
    g1 = bn_w.astype(jnp.float32) * gamma_eff
    g2 = bn_b.astype(jnp.float32) * gamma_eff + beta_eff
    g_pack = jnp.zeros((Cout, 8), jnp.float32)
    g_pack = g_pack.at[:, 0].set(g1).at[:, 1].set(g2)

    x_r = x.reshape(N, Cin, HW)
    w_cat = jnp.transpose(conv_w.astype(jnp.bfloat16), (2, 0, 3, 1)).reshape(
        3, Cout, 3 * Cin)

    BI = 8
    BO = 4
    n_conv = N // BI
    n_apply = N // BO
    kfn = functools.partial(
        _fused_kernel, cin=Cin, cout=Cout, h=H, w=W, bi=BI, bo=BO,
        n_conv=n_conv, inv_m=1.0 / float(M), eps=1e-5)
    out = pl.pallas_call(
        kfn,
        out_shape=jax.ShapeDtypeStruct((N, Cout, HW), jnp.float32),
        grid=(n_conv + n_apply,),
        in_specs=[
            pl.BlockSpec((BI, Cin, HW),
                         lambda t: (jnp.minimum(t, n_conv - 1), 0, 0)),
            pl.BlockSpec((3, Cout, 3 * Cin), lambda t: (0, 0, 0)),
            pl.BlockSpec((Cout, 8), lambda t: (0, 0)),
        ],
        out_specs=pl.BlockSpec(
            (BO, Cout, HW), lambda t: (jnp.maximum(t - n_conv, 0), 0, 0)),
        scratch_shapes=[
            pltpu.VMEM((N, Cout, HW), jnp.bfloat16),
            pltpu.VMEM((Cout, 8), jnp.float32),
        ],
        compiler_params=pltpu.CompilerParams(
            dimension_semantics=("arbitrary",)),
    )(x_r, w_cat, g_pack)

    return out.reshape(N, Cout, H, W)


# speedup vs baseline: 1.0199x; 1.0103x over previous
import functools

import jax
import jax.numpy as jnp
from jax.experimental import pallas as pl
from jax.experimental.pallas import tpu as pltpu


def _fused_kernel(x_ref, w_ref, g_ref, o_ref, conv_sc, st_sc, *,
                  cin, cout, h, w, bi, bo, n_conv, inv_m, eps):
    t = pl.program_id(0)
    hw = h * w

    @pl.when(t == 0)
    def _():
        st_sc[...] = jnp.zeros_like(st_sc)

    @pl.when(t < n_conv)
    def _():
        wmod = jax.lax.broadcasted_iota(jnp.int32, (1, hw), 1) % w
        zcol = jnp.zeros((cin, 1), jnp.bfloat16)
        zrow = jnp.zeros((cout, w), jnp.float32)
        tsum = jnp.zeros((cout, 1), jnp.float32)
        tsq = jnp.zeros((cout, 1), jnp.float32)
        for i in range(bi):
            x = x_ref[i].astype(jnp.bfloat16)
            xr = jnp.where(wmod != 0,
                           jnp.concatenate([zcol, x[:, : hw - 1]], axis=1), 0)
            xl = jnp.where(wmod != w - 1,
                           jnp.concatenate([x[:, 1:], zcol], axis=1), 0)
            xw = jnp.concatenate([xr, x, xl], axis=0)

            p0 = jnp.dot(w_ref[0], xw, preferred_element_type=jnp.float32)
            p1 = jnp.dot(w_ref[1], xw, preferred_element_type=jnp.float32)
            p2 = jnp.dot(w_ref[2], xw, preferred_element_type=jnp.float32)

            acc = p1
            acc = acc + jnp.concatenate([zrow, p0[:, : hw - w]], axis=1)
            acc = acc + jnp.concatenate([p2[:, w:], zrow], axis=1)

            conv_sc[t * bi + i] = acc.astype(jnp.bfloat16)
            tsum = tsum + jnp.sum(acc, axis=1, keepdims=True)
            tsq = tsq + jnp.sum(acc * acc, axis=1, keepdims=True)
        st_sc[:, 0:1] = st_sc[:, 0:1] + tsum
        st_sc[:, 1:2] = st_sc[:, 1:2] + tsq

    @pl.when(t >= n_conv)
    def _():
        b = t - n_conv
        st = st_sc[...]
        mean = st[:, 0:1] * inv_m
        var = jnp.maximum(st[:, 1:2] * inv_m - mean * mean, 0.0)
        inv_std = jax.lax.rsqrt(var + eps)
        g = g_ref[...]
        scale = g[:, 0:1] * inv_std
        shift = g[:, 1:2] - mean * scale
        for i in range(bo):
            c = conv_sc[b * bo + i].astype(jnp.float32)
            o_ref[i] = jnp.maximum(c * scale + shift, 0.0)


@jax.jit
def kernel(x, conv_w, conv_b, gamma, beta, bn_w, bn_b, A_taskpair):
    del conv_b
    N, Cin, H, W = x.shape
    Cout = conv_w.shape[0]
    HW = H * W
    M = N * HW

    A = A_taskpair.astype(jnp.float32)
    gamma_eff = (A @ gamma.astype(jnp.float32).T).reshape(Cout)
    beta_eff = (A @ beta.astype(jnp.float32).T).reshape(Cout)
    g1 = bn_w.astype(jnp.float32) * gamma_eff
    g2 = bn_b.astype(jnp.float32) * gamma_eff + beta_eff
    g_pack = jnp.zeros((Cout, 8), jnp.float32)
    g_pack = g_pack.at[:, 0].set(g1).at[:, 1].set(g2)

    x_r = x.reshape(N, Cin, HW)
    w_cat = jnp.transpose(conv_w.astype(jnp.bfloat16), (2, 0, 3, 1)).reshape(
        3, Cout, 3 * Cin)

    BI = 8
    BO = 8
    n_conv = N // BI
    n_apply = N // BO
    kfn = functools.partial(
        _fused_kernel, cin=Cin, cout=Cout, h=H, w=W, bi=BI, bo=BO,
        n_conv=n_conv, inv_m=1.0 / float(M), eps=1e-5)
    out = pl.pallas_call(
        kfn,
        out_shape=jax.ShapeDtypeStruct((N, Cout, HW), jnp.float32),
        grid=(n_conv + n_apply,),
        in_specs=[
            pl.BlockSpec((BI, Cin, HW),
                         lambda t: (jnp.minimum(t, n_conv - 1), 0, 0)),
            pl.BlockSpec((3, Cout, 3 * Cin), lambda t: (0, 0, 0)),
            pl.BlockSpec((Cout, 8), lambda t: (0, 0)),
        ],
        out_specs=pl.BlockSpec(
            (BO, Cout, HW), lambda t: (jnp.maximum(t - n_conv, 0), 0, 0)),
        scratch_shapes=[
            pltpu.VMEM((N, Cout, HW), jnp.bfloat16),
            pltpu.VMEM((Cout, 8), jnp.float32),
        ],
        compiler_params=pltpu.CompilerParams(
            dimension_semantics=("arbitrary",)),
    )(x_r, w_cat, g_pack)

    return out.reshape(N, Cout, H, W)
